# baseline (speedup 1.0000x reference)
import jax
import jax.numpy as jnp
from jax import lax
from jax.experimental import pallas as pl
from jax.experimental.pallas import tpu as pltpu
from jax.experimental.pallas import tpu_sc as plsc

N = 100000
E = 800000
NS = 16
CH = 128
EP = E // NS
NF = EP // CH
REM = EP - NF * CH
STRIPE = N // NS
FS = 6256
FL = N - (NS - 1) * FS
ZR = 250

ECP = E // 2 // NS
CNF = ECP // CH
CREM = ECP - CNF * CH

_SC_PARAMS = pltpu.CompilerParams(use_tc_tiling_on_sc=False)


def _zero_acc(z16, zbuf, acc, s):
  pltpu.sync_copy(z16, zbuf)
  for k in range(STRIPE // ZR):
    pltpu.sync_copy(zbuf, acc.at[pl.ds(s * STRIPE + k * ZR, ZR)])


def _flush_acc(acc, out, c, s):
  @pl.when(s < NS - 1)
  def _flush_main():
    fstripe = pl.ds(s * FS, FS)
    pltpu.sync_copy(acc.at[fstripe], out.at[fstripe, c])

  @pl.when(s == NS - 1)
  def _flush_tail():
    fstripe = pl.ds((NS - 1) * FS, FL)
    pltpu.sync_copy(acc.at[fstripe], out.at[fstripe, c])


ROWS = E // CH
RPT = ROWS // NS
K = 4
NP = RPT // (2 * K)
LEFT = RPT - NP * 2 * K
TAIL = ROWS - NS * RPT


def _make_layer(do_count):
  mesh = plsc.VectorSubcoreMesh(core_axis_name="c", subcore_axis_name="s")
  out_type = [jax.ShapeDtypeStruct((N, 2, 16), jnp.float32),
              jax.ShapeDtypeStruct((N, 2, 16), jnp.float32)]
  if do_count:
    out_type.append(jax.ShapeDtypeStruct((N, 2, 16), jnp.float32))
  scratch = [
      pltpu.VMEM_SHARED((N, 16), jnp.float32),
      pltpu.VMEM((2, K, CH), jnp.int32),
      pltpu.VMEM((2, K, CH), jnp.int32),
      pltpu.VMEM((2, K, CH), jnp.int32),
      pltpu.VMEM((2, K, CH, 16), jnp.float32),
      pltpu.VMEM((ZR, 16), jnp.float32),
      pltpu.VMEM((CH, 16), jnp.float32),
      pltpu.SemaphoreType.DMA,
      pltpu.SemaphoreType.DMA,
      pltpu.SemaphoreType.DMA,
  ]

  def body(*refs):
    if do_count:
      (table, src_tp3, dst_tp3, src_in3, dst_in3, z16, o16,
       out_tp, out_in, out_cnt,
       acc, srcb, gidx, dstb, rows, zbuf, onesb, semg, semsc, semz) = refs
    else:
      (table, src_tp3, dst_tp3, src_in3, dst_in3, z16, o16,
       out_tp, out_in,
       acc, srcb, gidx, dstb, rows, zbuf, onesb, semg, semsc, semz) = refs

    c = lax.axis_index("c")
    s = lax.axis_index("s")

    pltpu.sync_copy(o16, onesb)

    def zero_acc():
      pltpu.sync_copy(z16, zbuf)
      zd = [pltpu.async_copy(zbuf, acc.at[pl.ds(s * STRIPE + k * ZR, ZR)],
                             semz) for k in range(STRIPE // ZR)]
      for d in zd:
        d.wait()

    def load_idx(src3, dst3, roff, u, nrows):
      pltpu.sync_copy(src3.at[pl.ds(roff, nrows)],
                      srcb.at[u, pl.ds(0, nrows)])
      pltpu.sync_copy(dst3.at[pl.ds(roff, nrows)],
                      dstb.at[u, pl.ds(0, nrows)])
      for j in range(nrows):
        for v in range(CH // 16):
          sl = pl.ds(v * 16, 16)
          gidx[u, j, sl] = srcb[u, j, sl] * 2 + c

    def fire_gathers(u, nrows):
      return [pltpu.async_copy(table.at[gidx.at[u, j]], rows.at[u, j], semg)
              for j in range(nrows)]

    def fire_scatters(u, nrows):
      return [pltpu.async_copy(rows.at[u, j], acc.at[dstb.at[u, j]], semsc,
                               add=True) for j in range(nrows)]

    def scatter_group(src3, dst3, roff, nrows):
      load_idx(src3, dst3, roff, 0, nrows)
      for d in fire_gathers(0, nrows):
        d.wait()
      for d in fire_scatters(0, nrows):
        d.wait()

    def scatter_phase(src3, dst3, out):
      zero_acc()
      plsc.subcore_barrier()
      rowbase = s * RPT

      def loop_body(g, carry):
        r0 = rowbase + g * 2 * K
        load_idx(src3, dst3, r0, 0, K)
        for d in fire_gathers(0, K):
          d.wait()
        sa = fire_scatters(0, K)
        load_idx(src3, dst3, r0 + K, 1, K)
        for d in fire_gathers(1, K):
          d.wait()
        for d in sa:
          d.wait()
        for d in fire_scatters(1, K):
          d.wait()
        return carry

      lax.fori_loop(0, NP, loop_body, 0)
      scatter_group(src3, dst3, rowbase + NP * 2 * K, K)
      scatter_group(src3, dst3, rowbase + NP * 2 * K + K, LEFT - K)

      @pl.when(s < TAIL)
      def _tail():
        scatter_group(src3, dst3, NS * RPT + s, 1)

      plsc.subcore_barrier()
      _flush_acc(acc, out, c, s)
      plsc.subcore_barrier()

    def count_group(dst3, roff, nrows):
      pltpu.sync_copy(dst3.at[pl.ds(roff, nrows)],
                      dstb.at[0, pl.ds(0, nrows)])
      sd = [pltpu.async_copy(onesb, acc.at[dstb.at[0, j]], semsc, add=True)
            for j in range(nrows)]
      for d in sd:
        d.wait()

    def count_phase(dst3, out):
      zero_acc()
      plsc.subcore_barrier()
      rowbase = s * RPT

      def loop_body(g, carry):
        pltpu.sync_copy(dst3.at[pl.ds(rowbase + g * 2 * K, K)],
                        dstb.at[0, pl.ds(0, K)])
        sa = [pltpu.async_copy(onesb, acc.at[dstb.at[0, j]], semsc, add=True)
              for j in range(K)]
        pltpu.sync_copy(dst3.at[pl.ds(rowbase + g * 2 * K + K, K)],
                        dstb.at[1, pl.ds(0, K)])
        sb = [pltpu.async_copy(onesb, acc.at[dstb.at[1, j]], semsc, add=True)
              for j in range(K)]
        for d in sa + sb:
          d.wait()
        return carry

      lax.fori_loop(0, NP, loop_body, 0)
      count_group(dst3, rowbase + NP * 2 * K, K)
      count_group(dst3, rowbase + NP * 2 * K + K, LEFT - K)

      @pl.when(s < TAIL)
      def _tail():
        count_group(dst3, NS * RPT + s, 1)

      plsc.subcore_barrier()
      _flush_acc(acc, out, c, s)

    scatter_phase(src_tp3, dst_tp3, out_tp)
    scatter_phase(src_in3, dst_in3, out_in)
    if do_count:
      count_phase(dst_in3, out_cnt)

  return pl.kernel(body, out_type=out_type, mesh=mesh, scratch_types=scratch,
                   compiler_params=_SC_PARAMS,
                   name="sc_layer_cnt" if do_count else "sc_layer")


_layer1 = _make_layer(True)
_layer2 = _make_layer(False)

N4 = N // 4
BM4 = 2048


def _dense_common(x_ref, stp_ref, sin_ref, cnt_ref, wt_ref, wi_ref, wr_ref,
                  b_ref):
  x = x_ref[...]
  a = jnp.dot(stp_ref[...], wt_ref[...], preferred_element_type=jnp.float32,
              precision=jax.lax.Precision.HIGHEST)
  m = jnp.dot(sin_ref[...], wi_ref[...], preferred_element_type=jnp.float32,
              precision=jax.lax.Precision.HIGHEST)
  rec = 1.0 / jnp.maximum(cnt_ref[...], 1.0)
  r = jnp.dot(x, wr_ref[...], preferred_element_type=jnp.float32,
              precision=jax.lax.Precision.HIGHEST)
  return jnp.maximum(a + m * rec + r + b_ref[...], 0.0) + x


def _dense_body(x_ref, stp_ref, sin_ref, cnt_ref, wt_ref, wi_ref, wr_ref,
                b_ref, o_ref):
  o_ref[...] = _dense_common(x_ref, stp_ref, sin_ref, cnt_ref, wt_ref,
                             wi_ref, wr_ref, b_ref)


def _dense_cls_body(x_ref, stp_ref, sin_ref, cnt_ref, wt_ref, wi_ref, wr_ref,
                    b_ref, wc_ref, bc_ref, o_ref):
  h = _dense_common(x_ref, stp_ref, sin_ref, cnt_ref, wt_ref, wi_ref, wr_ref,
                    b_ref)
  o_ref[...] = jnp.dot(h, wc_ref[...],
                       preferred_element_type=jnp.float32,
              precision=jax.lax.Precision.HIGHEST) + bc_ref[...]


def _dense_call(cls):
  grid = (pl.cdiv(N4, BM4),)
  in_specs = [
      pl.BlockSpec((BM4, 128), lambda i: (i, 0)),
      pl.BlockSpec((BM4, 128), lambda i: (i, 0)),
      pl.BlockSpec((BM4, 128), lambda i: (i, 0)),
      pl.BlockSpec((BM4, 128), lambda i: (i, 0)),
      pl.BlockSpec((128, 128), lambda i: (0, 0)),
      pl.BlockSpec((128, 128), lambda i: (0, 0)),
      pl.BlockSpec((128, 128), lambda i: (0, 0)),
      pl.BlockSpec((1, 128), lambda i: (0, 0)),
  ]
  if cls:
    in_specs += [
        pl.BlockSpec((128, 32), lambda i: (0, 0)),
        pl.BlockSpec((1, 32), lambda i: (0, 0)),
    ]
    out_spec = pl.BlockSpec((BM4, 32), lambda i: (i, 0))
    out_shape = jax.ShapeDtypeStruct((N4, 32), jnp.float32)
    body = _dense_cls_body
  else:
    out_spec = pl.BlockSpec((BM4, 128), lambda i: (i, 0))
    out_shape = jax.ShapeDtypeStruct((N4, 128), jnp.float32)
    body = _dense_body
  return pl.pallas_call(body, grid=grid, in_specs=in_specs,
                        out_specs=out_spec, out_shape=out_shape)


_dense1 = _dense_call(False)
_dense2 = _dense_call(True)


def _kron4(w):
  return jnp.kron(jnp.eye(4, dtype=jnp.float32), w)


def kernel(x_stroke, edge_index_temp_previous, edge_index_intersects,
           Wm_tp1, Wr_tp1, b_tp1, Wm_in1, Wr_in1, b_in1,
           Wm_tp2, Wr_tp2, b_tp2, Wm_in2, Wr_in2, b_in2,
           W_cls, b_cls):
  src_tp3 = edge_index_temp_previous[0].reshape(ROWS, CH)
  dst_tp3 = edge_index_temp_previous[1].reshape(ROWS, CH)
  src_in3 = edge_index_intersects[0].reshape(ROWS, CH)
  dst_in3 = edge_index_intersects[1].reshape(ROWS, CH)
  z16 = jnp.zeros((ZR, 16), jnp.float32)
  o16 = jnp.ones((CH, 16), jnp.float32)

  table1 = x_stroke.reshape(2 * N, 16)
  stp1, sin1, cnt3 = _layer1(table1, src_tp3, dst_tp3, src_in3, dst_in3,
                             z16, o16)
  cnt4 = cnt3.reshape(N4, 128)

  x4 = x_stroke.reshape(N4, 128)
  h1 = _dense1(x4, stp1.reshape(N4, 128), sin1.reshape(N4, 128), cnt4,
               _kron4(Wm_tp1), _kron4(Wm_in1), _kron4(Wr_tp1 + Wr_in1),
               jnp.tile(b_tp1 + b_in1, 4).reshape(1, 128))

  table2 = h1.reshape(2 * N, 16)
  stp2, sin2 = _layer2(table2, src_tp3, dst_tp3, src_in3, dst_in3, z16, o16)

  wc = jnp.zeros((32, 8), jnp.float32).at[:, :7].set(W_cls)
  bc = jnp.zeros((8,), jnp.float32).at[:7].set(b_cls)
  out4 = _dense2(h1, stp2.reshape(N4, 128), sin2.reshape(N4, 128), cnt4,
                 _kron4(Wm_tp2), _kron4(Wm_in2), _kron4(Wr_tp2 + Wr_in2),
                 jnp.tile(b_tp2 + b_in2, 4).reshape(1, 128),
                 _kron4(wc), jnp.tile(bc, 4).reshape(1, 32))
  return out4.reshape(N, 8)[:, :7]
